# baseline (speedup 1.0000x reference)
import functools

import jax
import jax.numpy as jnp
from jax import lax
from jax.experimental import pallas as pl
from jax.experimental.pallas import tpu as pltpu
from jax.experimental.pallas import tpu_sc as plsc

_LAMBDA_SSI = 0.5
_TOP_K_MASKING = 0.1
_EPS = 1e-06

_B = 16
_N = 384 * 384
_HALF = _N // 2
_NVEC = _HALF // 16
_C = 18432
_NCHUNK = _N // _C

_SHIFTS = (23, 15, 7, 0)
_NBINS = (256, 256, 256, 128)


def _body_a(p_ref, t_ref, d_ref, kidx_ref, nv_ref, nv_scr):
    i = pl.program_id(0)
    p = p_ref[...]
    t = t_ref[...]
    mask = (t > _EPS)
    d = jnp.log(jnp.maximum(p, _EPS)) - jnp.log(jnp.maximum(t, _EPS))
    d = jnp.where(mask, d, 0.0)
    d_ref[...] = d

    @pl.when(i == 0)
    def _init():
        nv_scr[...] = jnp.zeros_like(nv_scr)

    nv_scr[...] += jnp.sum(mask.astype(jnp.float32), axis=1, keepdims=True)

    @pl.when(i == _NCHUNK - 1)
    def _final():
        nv = nv_scr[...]
        k = (nv * _TOP_K_MASKING).astype(jnp.int32)
        k = jnp.minimum(k, _N - 1)
        kidx_ref[...] = jnp.broadcast_to(k, (_B, 8))
        nv_ref[...] = jnp.broadcast_to(nv, (_B, 8))


def _sc_body(d_hbm, kidx_hbm, out_hbm, d_buf, hist, pbuf, xveci, state,
             kbuf, stbuf, sh_hist, sh_x):
    c = lax.axis_index("c")
    s = lax.axis_index("s")
    row = c * 8 + s // 2
    half = s % 2
    is_scan = half == 0
    lanes = lax.iota(jnp.int32, 16)
    ones = jnp.ones((16,), jnp.float32)

    pltpu.sync_copy(d_hbm.at[pl.ds(row * _N + half * _HALF, _HALF)], d_buf)

    @pl.when(is_scan)
    def _load_k():
        pltpu.sync_copy(kidx_hbm.at[pl.ds(row * 8, 8)], kbuf.at[pl.ds(0, 8)])
        state[...] = jnp.full((16,), kbuf[...][0].astype(jnp.float32))

    def load_u(i):
        d = d_buf[pl.ds(i * 16, 16)]
        u = lax.bitcast_convert_type(d, jnp.int32) & 0x7FFFFFFF
        return d, u

    for p in range(4):
        shift = _SHIFTS[p]
        nb = _NBINS[p]

        def zbody(j, carry):
            hist[pl.ds(j * 16, 16)] = jnp.zeros((16,), jnp.float32)
            return carry
        lax.fori_loop(0, 256, zbody, 0)

        if p == 0:
            def sbody(i, carry):
                _, u = load_u(i)
                b = lax.shift_right_logical(u, 23)
                plsc.addupdate_scatter(hist, [b * 16 + lanes], ones)
                return carry
        else:
            pfx = xveci[...][0]
            pr_shift = _SHIFTS[p - 1]

            def sbody(i, carry, pfx=pfx, shift=shift, pr_shift=pr_shift,
                      mb=nb - 1):
                _, u = load_u(i)
                m = lax.shift_right_logical(u, pr_shift) == pfx
                b = lax.shift_right_logical(u, shift) & mb
                plsc.addupdate_scatter(hist, [b * 16 + lanes], ones, mask=m)
                return carry
        lax.fori_loop(0, _NVEC, sbody, 0)

        pltpu.sync_copy(hist, sh_hist.at[pl.ds(s * 4096, 4096)])
        plsc.subcore_barrier()

        @pl.when(is_scan)
        def _scan(p=p, nb=nb):
            pltpu.sync_copy(sh_hist.at[pl.ds((s + 1) * 4096, 4096)], pbuf)
            rank = state[...][0]

            def scbody(j, carry):
                found, bf, cgt, rn = carry
                b = (nb - 1) - j
                cb = (jnp.sum(hist[pl.ds(b * 16, 16)]) +
                      jnp.sum(pbuf[pl.ds(b * 16, 16)]))
                take = jnp.logical_and(found == 0, cgt + cb > rank)
                bf = jnp.where(take, b, bf)
                rn = jnp.where(take, rank - cgt, rn)
                found = jnp.where(take, 1, found)
                cgt = jnp.where(found > 0, cgt, cgt + cb)
                return found, bf, cgt, rn

            init = (jnp.int32(0), jnp.int32(0), jnp.float32(0.0),
                    jnp.float32(0.0))
            _, bf, _, rn = lax.fori_loop(0, nb, scbody, init)
            state[...] = jnp.full((16,), rn)
            if p == 0:
                pfx_new = bf
            elif p < 3:
                pfx_new = xveci[...][0] * 256 + bf
            else:
                pfx_new = xveci[...][0] * 128 + bf
            xveci[...] = jnp.full((16,), pfx_new)
            pltpu.sync_copy(xveci, sh_x.at[pl.ds(s * 16, 16)])

        plsc.subcore_barrier()

        @pl.when(jnp.logical_not(is_scan))
        def _recv():
            pltpu.sync_copy(sh_x.at[pl.ds((s - 1) * 16, 16)], xveci)

    thr = xveci[...][0]

    def fbody(i, carry):
        sv, ssv, cxv = carry
        d, u = load_u(i)
        m = u < thr
        dm = jnp.where(m, d, 0.0)
        return sv + dm, ssv + dm * dm, cxv + jnp.where(m, 0.0, 1.0)

    z = jnp.zeros((16,), jnp.float32)
    sv, ssv, cxv = lax.fori_loop(0, _NVEC, fbody, (z, z, z))
    stbuf[pl.ds(0, 16)] = sv
    stbuf[pl.ds(16, 16)] = ssv
    stbuf[pl.ds(32, 16)] = cxv
    pltpu.sync_copy(stbuf, out_hbm.at[pl.ds(row * 96 + half * 48, 48)])


@functools.cache
def _sc_select():
    return pl.kernel(
        _sc_body,
        out_type=jax.ShapeDtypeStruct((_B * 96,), jnp.float32),
        mesh=plsc.VectorSubcoreMesh(core_axis_name="c", subcore_axis_name="s",
                                    num_cores=2, num_subcores=16),
        compiler_params=pltpu.CompilerParams(needs_layout_passes=False),
        scratch_types=[
        pltpu.VMEM((_HALF,), jnp.float32),
        pltpu.VMEM((4096,), jnp.float32),
        pltpu.VMEM((4096,), jnp.float32),
        pltpu.VMEM((16,), jnp.int32),
        pltpu.VMEM((16,), jnp.float32),
        pltpu.VMEM((16,), jnp.int32),
        pltpu.VMEM((48,), jnp.float32),
            pltpu.VMEM_SHARED((65536,), jnp.float32),
            pltpu.VMEM_SHARED((256,), jnp.int32),
        ],
    )


def _body_c(stats_ref, nv_ref, out_ref):
    st = stats_ref[...]
    s = jnp.sum(st[:, 0:16] + st[:, 48:64], axis=1, keepdims=True)
    ss = jnp.sum(st[:, 16:32] + st[:, 64:80], axis=1, keepdims=True)
    cex = jnp.sum(st[:, 32:48] + st[:, 80:96], axis=1, keepdims=True)
    nv = nv_ref[:, 0:1]
    n = jnp.maximum(nv - cex, 1.0)
    v = ss / n - _LAMBDA_SSI * (s * s) / (n * n)
    row = jnp.sqrt(jnp.maximum(v, _EPS))
    out_ref[...] = jnp.mean(row).reshape(1, 1)


def kernel(prediction, target):
    p = prediction.reshape(_B, _N)
    t = target.reshape(_B, _N)
    d, kidx, nv = pl.pallas_call(
        _body_a,
        grid=(_NCHUNK,),
        in_specs=[
            pl.BlockSpec((_B, _C), lambda i: (0, i)),
            pl.BlockSpec((_B, _C), lambda i: (0, i)),
        ],
        out_specs=[
            pl.BlockSpec((_B, _C), lambda i: (0, i)),
            pl.BlockSpec((_B, 8), lambda i: (0, 0)),
            pl.BlockSpec((_B, 8), lambda i: (0, 0)),
        ],
        out_shape=[
            jax.ShapeDtypeStruct((_B, _N), jnp.float32),
            jax.ShapeDtypeStruct((_B, 8), jnp.int32),
            jax.ShapeDtypeStruct((_B, 8), jnp.float32),
        ],
        scratch_shapes=[pltpu.VMEM((_B, 1), jnp.float32)],
    )(p, t)

    stats = _sc_select()(d.reshape(-1), kidx.reshape(-1)).reshape(_B, 96)

    out = pl.pallas_call(
        _body_c,
        out_specs=pl.BlockSpec((1, 1), lambda: (0, 0)),
        out_shape=jax.ShapeDtypeStruct((1, 1), jnp.float32),
    )(stats, nv)
    return out[0, 0]


# speedup vs baseline: 1.1922x; 1.1922x over previous
import functools

import jax
import jax.numpy as jnp
from jax import lax
from jax.experimental import pallas as pl
from jax.experimental.pallas import tpu as pltpu
from jax.experimental.pallas import tpu_sc as plsc

_LAMBDA_SSI = 0.5
_TOP_K_MASKING = 0.1
_EPS = 1e-06

_B = 16
_N = 384 * 384
_HALF = _N // 2
_NVEC = _HALF // 16
_C = 18432
_NCHUNK = _N // _C

_SHIFTS = (23, 15, 7, 0)
_NBINS = (256, 256, 256, 128)


def _body_a(p_ref, t_ref, d_ref, kidx_ref, nv_ref, nv_scr):
    i = pl.program_id(0)
    p = p_ref[...]
    t = t_ref[...]
    mask = (t > _EPS)
    d = jnp.log(jnp.maximum(p, _EPS)) - jnp.log(jnp.maximum(t, _EPS))
    d = jnp.where(mask, d, 0.0)
    d_ref[...] = d

    @pl.when(i == 0)
    def _init():
        nv_scr[...] = jnp.zeros_like(nv_scr)

    nv_scr[...] += jnp.sum(mask.astype(jnp.float32), axis=1, keepdims=True)

    @pl.when(i == _NCHUNK - 1)
    def _final():
        nv = nv_scr[...]
        k = (nv * _TOP_K_MASKING).astype(jnp.int32)
        k = jnp.minimum(k, _N - 1)
        kidx_ref[...] = jnp.broadcast_to(k, (_B, 8))
        nv_ref[...] = jnp.broadcast_to(nv, (_B, 8))


def _sc_body(d_hbm, kidx_hbm, out_hbm, d_buf, hist, pbuf, xveci, state,
             kbuf, stbuf, sh_hist, sh_x):
    c = lax.axis_index("c")
    s = lax.axis_index("s")
    row = c * 8 + s // 2
    half = s % 2
    is_scan = half == 0
    lanes = lax.iota(jnp.int32, 16)
    ones = jnp.ones((16,), jnp.float32)

    pltpu.sync_copy(d_hbm.at[row, pl.ds(half * _HALF, _HALF)], d_buf)

    @pl.when(is_scan)
    def _load_k():
        pltpu.sync_copy(kidx_hbm.at[pl.ds(row * 8, 8)], kbuf.at[pl.ds(0, 8)])
        state[...] = jnp.full((16,), kbuf[...][0].astype(jnp.float32))

    def load_u(i):
        d = d_buf[pl.ds(i * 16, 16)]
        u = lax.bitcast_convert_type(d, jnp.int32) & 0x7FFFFFFF
        return d, u

    def unrolled(n, fn, unroll=8):
        def body(j, carry):
            for u in range(unroll):
                fn(j * unroll + u)
            return carry
        lax.fori_loop(0, n // unroll, body, 0)

    for p in range(4):
        shift = _SHIFTS[p]
        nb = _NBINS[p]

        def zfn(j):
            hist[pl.ds(j * 16, 16)] = jnp.zeros((16,), jnp.float32)
        unrolled(256, zfn)

        if p == 0:
            def sfn(i):
                _, u = load_u(i)
                b = lax.shift_right_logical(u, 23)
                plsc.addupdate_scatter(hist, [b * 16 + lanes], ones)
        else:
            pfx = xveci[...][0]
            pr_shift = _SHIFTS[p - 1]

            def sfn(i, pfx=pfx, shift=shift, pr_shift=pr_shift, mb=nb - 1):
                _, u = load_u(i)
                m = lax.shift_right_logical(u, pr_shift) == pfx
                b = lax.shift_right_logical(u, shift) & mb
                plsc.addupdate_scatter(hist, [b * 16 + lanes], ones, mask=m)
        unrolled(_NVEC, sfn)

        pltpu.sync_copy(hist, sh_hist.at[pl.ds(s * 4096, 4096)])
        plsc.subcore_barrier()

        @pl.when(is_scan)
        def _scan(p=p, nb=nb):
            pltpu.sync_copy(sh_hist.at[pl.ds((s + 1) * 4096, 4096)], pbuf)
            rank = state[...][0]

            def scbody(j, carry):
                found, bf, cgt, rn = carry
                b = (nb - 1) - j
                cb = (jnp.sum(hist[pl.ds(b * 16, 16)]) +
                      jnp.sum(pbuf[pl.ds(b * 16, 16)]))
                take = jnp.logical_and(found == 0, cgt + cb > rank)
                bf = jnp.where(take, b, bf)
                rn = jnp.where(take, rank - cgt, rn)
                found = jnp.where(take, 1, found)
                cgt = jnp.where(found > 0, cgt, cgt + cb)
                return found, bf, cgt, rn

            init = (jnp.int32(0), jnp.int32(0), jnp.float32(0.0),
                    jnp.float32(0.0))
            _, bf, _, rn = lax.fori_loop(0, nb, scbody, init)
            state[...] = jnp.full((16,), rn)
            if p == 0:
                pfx_new = bf
            elif p < 3:
                pfx_new = xveci[...][0] * 256 + bf
            else:
                pfx_new = xveci[...][0] * 128 + bf
            xveci[...] = jnp.full((16,), pfx_new)
            pltpu.sync_copy(xveci, sh_x.at[pl.ds(s * 16, 16)])

        plsc.subcore_barrier()

        @pl.when(jnp.logical_not(is_scan))
        def _recv():
            pltpu.sync_copy(sh_x.at[pl.ds((s - 1) * 16, 16)], xveci)

    thr = xveci[...][0]

    def fbody(j, carry):
        accs = list(carry)
        for u8 in range(8):
            i = j * 8 + u8
            d, u = load_u(i)
            m = u < thr
            dm = jnp.where(m, d, 0.0)
            a = accs[u8 % 4]
            accs[u8 % 4] = (a[0] + dm, a[1] + dm * dm,
                            a[2] + jnp.where(m, 0.0, 1.0))
        return tuple(accs)

    z = jnp.zeros((16,), jnp.float32)
    accs = lax.fori_loop(0, _NVEC // 8, fbody, ((z, z, z),) * 4)
    sv = accs[0][0] + accs[1][0] + accs[2][0] + accs[3][0]
    ssv = accs[0][1] + accs[1][1] + accs[2][1] + accs[3][1]
    cxv = accs[0][2] + accs[1][2] + accs[2][2] + accs[3][2]
    stbuf[pl.ds(0, 16)] = sv
    stbuf[pl.ds(16, 16)] = ssv
    stbuf[pl.ds(32, 16)] = cxv
    pltpu.sync_copy(stbuf, out_hbm.at[pl.ds(row * 96 + half * 48, 48)])


@functools.cache
def _sc_select():
    return pl.kernel(
        _sc_body,
        out_type=jax.ShapeDtypeStruct((_B * 96,), jnp.float32),
        mesh=plsc.VectorSubcoreMesh(core_axis_name="c", subcore_axis_name="s",
                                    num_cores=2, num_subcores=16),
        compiler_params=pltpu.CompilerParams(needs_layout_passes=False),
        scratch_types=[
        pltpu.VMEM((_HALF,), jnp.float32),
        pltpu.VMEM((4096,), jnp.float32),
        pltpu.VMEM((4096,), jnp.float32),
        pltpu.VMEM((16,), jnp.int32),
        pltpu.VMEM((16,), jnp.float32),
        pltpu.VMEM((16,), jnp.int32),
        pltpu.VMEM((48,), jnp.float32),
            pltpu.VMEM_SHARED((65536,), jnp.float32),
            pltpu.VMEM_SHARED((256,), jnp.int32),
        ],
    )


def _body_c(stats_ref, nv_ref, out_ref):
    st = stats_ref[...]
    s = jnp.sum(st[:, 0:16] + st[:, 48:64], axis=1, keepdims=True)
    ss = jnp.sum(st[:, 16:32] + st[:, 64:80], axis=1, keepdims=True)
    cex = jnp.sum(st[:, 32:48] + st[:, 80:96], axis=1, keepdims=True)
    nv = nv_ref[:, 0:1]
    n = jnp.maximum(nv - cex, 1.0)
    v = ss / n - _LAMBDA_SSI * (s * s) / (n * n)
    row = jnp.sqrt(jnp.maximum(v, _EPS))
    out_ref[...] = jnp.mean(row).reshape(1, 1)


def kernel(prediction, target):
    p = prediction.reshape(_B, _N)
    t = target.reshape(_B, _N)
    d, kidx, nv = pl.pallas_call(
        _body_a,
        grid=(_NCHUNK,),
        in_specs=[
            pl.BlockSpec((_B, _C), lambda i: (0, i)),
            pl.BlockSpec((_B, _C), lambda i: (0, i)),
        ],
        out_specs=[
            pl.BlockSpec((_B, _C), lambda i: (0, i)),
            pl.BlockSpec((_B, 8), lambda i: (0, 0)),
            pl.BlockSpec((_B, 8), lambda i: (0, 0)),
        ],
        out_shape=[
            jax.ShapeDtypeStruct((_B, _N), jnp.float32),
            jax.ShapeDtypeStruct((_B, 8), jnp.int32),
            jax.ShapeDtypeStruct((_B, 8), jnp.float32),
        ],
        scratch_shapes=[pltpu.VMEM((_B, 1), jnp.float32)],
    )(p, t)

    stats = _sc_select()(d, kidx.reshape(-1)).reshape(_B, 96)

    out = pl.pallas_call(
        _body_c,
        out_specs=pl.BlockSpec((1, 1), lambda: (0, 0)),
        out_shape=jax.ShapeDtypeStruct((1, 1), jnp.float32),
    )(stats, nv)
    return out[0, 0]


# speedup vs baseline: 2.6635x; 2.2340x over previous
import functools

import jax
import jax.numpy as jnp
from jax import lax
from jax.experimental import pallas as pl
from jax.experimental.pallas import tpu as pltpu
from jax.experimental.pallas import tpu_sc as plsc

_LAMBDA_SSI = 0.5
_TOP_K_MASKING = 0.1
_EPS = 1e-06

_B = 16
_N = 384 * 384
_HALF = _N // 2
_NVEC = _HALF // 16
_C = 18432
_NCHUNK = _N // _C

_SHIFTS = (23, 15, 7, 0)
_NBINS = (256, 256, 256, 128)


def _body_a(p_ref, t_ref, d_ref, kidx_ref, nv_ref, nv_scr):
    i = pl.program_id(0)
    p = p_ref[...]
    t = t_ref[...]
    mask = (t > _EPS)
    d = jnp.log(jnp.maximum(p, _EPS)) - jnp.log(jnp.maximum(t, _EPS))
    d = jnp.where(mask, d, 0.0)
    d_ref[...] = d

    @pl.when(i == 0)
    def _init():
        nv_scr[...] = jnp.zeros_like(nv_scr)

    nv_scr[...] += jnp.sum(mask.astype(jnp.float32), axis=1, keepdims=True)

    @pl.when(i == _NCHUNK - 1)
    def _final():
        nv = nv_scr[...]
        k = (nv * _TOP_K_MASKING).astype(jnp.int32)
        k = jnp.minimum(k, _N - 1)
        kidx_ref[...] = jnp.broadcast_to(k, (_B, 8))
        nv_ref[...] = jnp.broadcast_to(nv, (_B, 8))


def _sc_body(d_hbm, kidx_hbm, out_hbm, d_buf, hist, pbuf, xveci, state,
             kbuf, stbuf, sh_hist, sh_x):
    c = lax.axis_index("c")
    s = lax.axis_index("s")
    row = c * 8 + s // 2
    half = s % 2
    is_scan = half == 0
    lanes = lax.iota(jnp.int32, 16)
    ones = jnp.ones((16,), jnp.float32)

    pltpu.sync_copy(d_hbm.at[row, pl.ds(half * _HALF, _HALF)], d_buf)

    @pl.when(is_scan)
    def _load_k():
        pltpu.sync_copy(kidx_hbm.at[pl.ds(row * 8, 8)], kbuf.at[pl.ds(0, 8)])
        state[...] = jnp.full((16,), kbuf[...][0].astype(jnp.float32))

    def load_u(i):
        d = d_buf[pl.ds(i * 16, 16)]
        u = lax.bitcast_convert_type(d, jnp.int32) & 0x7FFFFFFF
        return d, u

    def unrolled(n, fn, unroll=8):
        plsc.parallel_loop(0, n, unroll=unroll)(fn)

    for p in range(4):
        shift = _SHIFTS[p]
        nb = _NBINS[p]

        def zfn(j):
            hist[pl.ds(j * 16, 16)] = jnp.zeros((16,), jnp.float32)
        unrolled(256, zfn)

        if p == 0:
            def sfn(i):
                _, u = load_u(i)
                b = lax.shift_right_logical(u, 23)
                plsc.addupdate_scatter(hist, [b * 16 + lanes], ones)
        else:
            pfx = xveci[...][0]
            pr_shift = _SHIFTS[p - 1]

            def sfn(i, pfx=pfx, shift=shift, pr_shift=pr_shift, mb=nb - 1):
                _, u = load_u(i)
                m = lax.shift_right_logical(u, pr_shift) == pfx
                b = lax.shift_right_logical(u, shift) & mb
                plsc.addupdate_scatter(hist, [b * 16 + lanes], ones, mask=m)
        unrolled(_NVEC, sfn)

        pltpu.sync_copy(hist, sh_hist.at[pl.ds(s * 4096, 4096)])
        plsc.subcore_barrier()

        @pl.when(is_scan)
        def _scan(p=p, nb=nb):
            pltpu.sync_copy(sh_hist.at[pl.ds((s + 1) * 4096, 4096)], pbuf)
            rank = state[...][0]

            init = (jnp.int32(0), jnp.int32(0), jnp.float32(0.0),
                    jnp.float32(0.0))

            @plsc.parallel_loop(0, nb, unroll=4, carry=init)
            def scbody(j, carry):
                found, bf, cgt, rn = carry
                b = (nb - 1) - j
                cb = (jnp.sum(hist[pl.ds(b * 16, 16)]) +
                      jnp.sum(pbuf[pl.ds(b * 16, 16)]))
                take = jnp.logical_and(found == 0, cgt + cb > rank)
                bf = jnp.where(take, b, bf)
                rn = jnp.where(take, rank - cgt, rn)
                found = jnp.where(take, 1, found)
                cgt = jnp.where(found > 0, cgt, cgt + cb)
                return found, bf, cgt, rn

            _, bf, _, rn = scbody
            state[...] = jnp.full((16,), rn)
            if p == 0:
                pfx_new = bf
            elif p < 3:
                pfx_new = xveci[...][0] * 256 + bf
            else:
                pfx_new = xveci[...][0] * 128 + bf
            xveci[...] = jnp.full((16,), pfx_new)
            pltpu.sync_copy(xveci, sh_x.at[pl.ds(s * 16, 16)])

        plsc.subcore_barrier()

        @pl.when(jnp.logical_not(is_scan))
        def _recv():
            pltpu.sync_copy(sh_x.at[pl.ds((s - 1) * 16, 16)], xveci)

    thr = xveci[...][0]

    z = jnp.zeros((16,), jnp.float32)

    @plsc.parallel_loop(0, _NVEC, unroll=8, carry=(z, z, z))
    def facc(i, carry):
        sv, ssv, cxv = carry
        d, u = load_u(i)
        m = u < thr
        dm = jnp.where(m, d, 0.0)
        return sv + dm, ssv + dm * dm, cxv + jnp.where(m, 0.0, 1.0)

    sv, ssv, cxv = facc
    stbuf[pl.ds(0, 16)] = sv
    stbuf[pl.ds(16, 16)] = ssv
    stbuf[pl.ds(32, 16)] = cxv
    pltpu.sync_copy(stbuf, out_hbm.at[pl.ds(row * 96 + half * 48, 48)])


@functools.cache
def _sc_select():
    return pl.kernel(
        _sc_body,
        out_type=jax.ShapeDtypeStruct((_B * 96,), jnp.float32),
        mesh=plsc.VectorSubcoreMesh(core_axis_name="c", subcore_axis_name="s",
                                    num_cores=2, num_subcores=16),
        compiler_params=pltpu.CompilerParams(needs_layout_passes=False),
        scratch_types=[
        pltpu.VMEM((_HALF,), jnp.float32),
        pltpu.VMEM((4096,), jnp.float32),
        pltpu.VMEM((4096,), jnp.float32),
        pltpu.VMEM((16,), jnp.int32),
        pltpu.VMEM((16,), jnp.float32),
        pltpu.VMEM((16,), jnp.int32),
        pltpu.VMEM((48,), jnp.float32),
            pltpu.VMEM_SHARED((65536,), jnp.float32),
            pltpu.VMEM_SHARED((256,), jnp.int32),
        ],
    )


def _body_c(stats_ref, nv_ref, out_ref):
    st = stats_ref[...]
    s = jnp.sum(st[:, 0:16] + st[:, 48:64], axis=1, keepdims=True)
    ss = jnp.sum(st[:, 16:32] + st[:, 64:80], axis=1, keepdims=True)
    cex = jnp.sum(st[:, 32:48] + st[:, 80:96], axis=1, keepdims=True)
    nv = nv_ref[:, 0:1]
    n = jnp.maximum(nv - cex, 1.0)
    v = ss / n - _LAMBDA_SSI * (s * s) / (n * n)
    row = jnp.sqrt(jnp.maximum(v, _EPS))
    out_ref[...] = jnp.mean(row).reshape(1, 1)


def kernel(prediction, target):
    p = prediction.reshape(_B, _N)
    t = target.reshape(_B, _N)
    d, kidx, nv = pl.pallas_call(
        _body_a,
        grid=(_NCHUNK,),
        in_specs=[
            pl.BlockSpec((_B, _C), lambda i: (0, i)),
            pl.BlockSpec((_B, _C), lambda i: (0, i)),
        ],
        out_specs=[
            pl.BlockSpec((_B, _C), lambda i: (0, i)),
            pl.BlockSpec((_B, 8), lambda i: (0, 0)),
            pl.BlockSpec((_B, 8), lambda i: (0, 0)),
        ],
        out_shape=[
            jax.ShapeDtypeStruct((_B, _N), jnp.float32),
            jax.ShapeDtypeStruct((_B, 8), jnp.int32),
            jax.ShapeDtypeStruct((_B, 8), jnp.float32),
        ],
        scratch_shapes=[pltpu.VMEM((_B, 1), jnp.float32)],
    )(p, t)

    stats = _sc_select()(d, kidx.reshape(-1)).reshape(_B, 96)

    out = pl.pallas_call(
        _body_c,
        out_specs=pl.BlockSpec((1, 1), lambda: (0, 0)),
        out_shape=jax.ShapeDtypeStruct((1, 1), jnp.float32),
    )(stats, nv)
    return out[0, 0]


# speedup vs baseline: 2.7343x; 1.0266x over previous
import functools

import jax
import jax.numpy as jnp
from jax import lax
from jax.experimental import pallas as pl
from jax.experimental.pallas import tpu as pltpu
from jax.experimental.pallas import tpu_sc as plsc

_LAMBDA_SSI = 0.5
_TOP_K_MASKING = 0.1
_EPS = 1e-06

_B = 16
_N = 384 * 384
_HALF = _N // 2
_NVEC = _HALF // 16
_C = 18432
_NCHUNK = _N // _C

_SHIFTS = (23, 15, 7, 0)
_NBINS = (256, 256, 256, 128)


def _body_a(p_ref, t_ref, d_ref, kidx_ref, nv_ref, nv_scr):
    i = pl.program_id(0)
    p = p_ref[...]
    t = t_ref[...]
    mask = (t > _EPS)
    d = jnp.log(jnp.maximum(p, _EPS)) - jnp.log(jnp.maximum(t, _EPS))
    d = jnp.where(mask, d, 0.0)
    d_ref[...] = d

    @pl.when(i == 0)
    def _init():
        nv_scr[...] = jnp.zeros_like(nv_scr)

    nv_scr[...] += jnp.sum(mask.astype(jnp.float32), axis=1, keepdims=True)

    @pl.when(i == _NCHUNK - 1)
    def _final():
        nv = nv_scr[...]
        k = (nv * _TOP_K_MASKING).astype(jnp.int32)
        k = jnp.minimum(k, _N - 1)
        kidx_ref[...] = jnp.broadcast_to(k, (_B, 128))
        nv_ref[...] = jnp.broadcast_to(nv, (_B, 8))


def _sc_body(d_hbm, kidx_hbm, out_hbm, d_buf, hist, pbuf, xveci, state,
             kbuf, stbuf, sh_hist, sh_x):
    c = lax.axis_index("c")
    s = lax.axis_index("s")
    row = c * 8 + s // 2
    half = s % 2
    is_scan = half == 0
    lanes = lax.iota(jnp.int32, 16)
    ones = jnp.ones((16,), jnp.float32)

    pltpu.sync_copy(d_hbm.at[row, pl.ds(half * _HALF, _HALF)], d_buf)

    @pl.when(is_scan)
    def _load_k():
        pltpu.sync_copy(kidx_hbm.at[row], kbuf)
        state[...] = jnp.full((16,), kbuf[pl.ds(0, 16)][0].astype(jnp.float32))

    def load_u(i):
        d = d_buf[pl.ds(i * 16, 16)]
        u = lax.bitcast_convert_type(d, jnp.int32) & 0x7FFFFFFF
        return d, u

    def unrolled(n, fn, unroll=8):
        plsc.parallel_loop(0, n, unroll=unroll)(fn)

    for p in range(4):
        shift = _SHIFTS[p]
        nb = _NBINS[p]

        def zfn(j):
            hist[pl.ds(j * 16, 16)] = jnp.zeros((16,), jnp.float32)
        unrolled(256, zfn)

        if p == 0:
            def sfn(i):
                _, u = load_u(i)
                b = lax.shift_right_logical(u, 23)
                plsc.addupdate_scatter(hist, [b * 16 + lanes], ones)
        else:
            pfx = xveci[...][0]
            pr_shift = _SHIFTS[p - 1]

            def sfn(i, pfx=pfx, shift=shift, pr_shift=pr_shift, mb=nb - 1):
                _, u = load_u(i)
                m = lax.shift_right_logical(u, pr_shift) == pfx
                b = lax.shift_right_logical(u, shift) & mb
                plsc.addupdate_scatter(hist, [b * 16 + lanes], ones, mask=m)
        unrolled(_NVEC, sfn)

        pltpu.sync_copy(hist, sh_hist.at[pl.ds(s * 4096, 4096)])
        plsc.subcore_barrier()

        @pl.when(is_scan)
        def _scan(p=p, nb=nb):
            pltpu.sync_copy(sh_hist.at[pl.ds((s + 1) * 4096, 4096)], pbuf)
            rank = state[...][0]

            init = (jnp.int32(0), jnp.int32(0), jnp.float32(0.0),
                    jnp.float32(0.0))

            @plsc.parallel_loop(0, nb, unroll=4, carry=init)
            def scbody(j, carry):
                found, bf, cgt, rn = carry
                b = (nb - 1) - j
                cb = (jnp.sum(hist[pl.ds(b * 16, 16)]) +
                      jnp.sum(pbuf[pl.ds(b * 16, 16)]))
                take = jnp.logical_and(found == 0, cgt + cb > rank)
                bf = jnp.where(take, b, bf)
                rn = jnp.where(take, rank - cgt, rn)
                found = jnp.where(take, 1, found)
                cgt = jnp.where(found > 0, cgt, cgt + cb)
                return found, bf, cgt, rn

            _, bf, _, rn = scbody
            state[...] = jnp.full((16,), rn)
            if p == 0:
                pfx_new = bf
            elif p < 3:
                pfx_new = xveci[...][0] * 256 + bf
            else:
                pfx_new = xveci[...][0] * 128 + bf
            xveci[...] = jnp.full((16,), pfx_new)
            pltpu.sync_copy(xveci, sh_x.at[pl.ds(s * 16, 16)])

        plsc.subcore_barrier()

        @pl.when(jnp.logical_not(is_scan))
        def _recv():
            pltpu.sync_copy(sh_x.at[pl.ds((s - 1) * 16, 16)], xveci)

    thr = xveci[...][0]

    z = jnp.zeros((16,), jnp.float32)

    @plsc.parallel_loop(0, _NVEC, unroll=8, carry=(z, z, z))
    def facc(i, carry):
        sv, ssv, cxv = carry
        d, u = load_u(i)
        m = u < thr
        dm = jnp.where(m, d, 0.0)
        return sv + dm, ssv + dm * dm, cxv + jnp.where(m, 0.0, 1.0)

    sv, ssv, cxv = facc
    stbuf[pl.ds(0, 16)] = sv
    stbuf[pl.ds(16, 16)] = ssv
    stbuf[pl.ds(32, 16)] = cxv
    pltpu.sync_copy(stbuf, out_hbm.at[half * 16 + row])


@functools.cache
def _sc_select():
    return pl.kernel(
        _sc_body,
        out_type=jax.ShapeDtypeStruct((2 * _B, 128), jnp.float32),
        mesh=plsc.VectorSubcoreMesh(core_axis_name="c", subcore_axis_name="s",
                                    num_cores=2, num_subcores=16),
        compiler_params=pltpu.CompilerParams(needs_layout_passes=False),
        scratch_types=[
        pltpu.VMEM((_HALF,), jnp.float32),
        pltpu.VMEM((4096,), jnp.float32),
        pltpu.VMEM((4096,), jnp.float32),
        pltpu.VMEM((16,), jnp.int32),
        pltpu.VMEM((16,), jnp.float32),
        pltpu.VMEM((128,), jnp.int32),
        pltpu.VMEM((128,), jnp.float32),
            pltpu.VMEM_SHARED((65536,), jnp.float32),
            pltpu.VMEM_SHARED((256,), jnp.int32),
        ],
    )


def _body_c(stats_ref, nv_ref, out_ref):
    st = stats_ref[0:_B, :] + stats_ref[_B:2 * _B, :]
    s = jnp.sum(st[:, 0:16], axis=1, keepdims=True)
    ss = jnp.sum(st[:, 16:32], axis=1, keepdims=True)
    cex = jnp.sum(st[:, 32:48], axis=1, keepdims=True)
    nv = nv_ref[:, 0:1]
    n = jnp.maximum(nv - cex, 1.0)
    v = ss / n - _LAMBDA_SSI * (s * s) / (n * n)
    row = jnp.sqrt(jnp.maximum(v, _EPS))
    out_ref[...] = jnp.mean(row).reshape(1, 1)


def kernel(prediction, target):
    p = prediction.reshape(_B, _N)
    t = target.reshape(_B, _N)
    d, kidx, nv = pl.pallas_call(
        _body_a,
        grid=(_NCHUNK,),
        in_specs=[
            pl.BlockSpec((_B, _C), lambda i: (0, i)),
            pl.BlockSpec((_B, _C), lambda i: (0, i)),
        ],
        out_specs=[
            pl.BlockSpec((_B, _C), lambda i: (0, i)),
            pl.BlockSpec((_B, 128), lambda i: (0, 0)),
            pl.BlockSpec((_B, 8), lambda i: (0, 0)),
        ],
        out_shape=[
            jax.ShapeDtypeStruct((_B, _N), jnp.float32),
            jax.ShapeDtypeStruct((_B, 128), jnp.int32),
            jax.ShapeDtypeStruct((_B, 8), jnp.float32),
        ],
        scratch_shapes=[pltpu.VMEM((_B, 1), jnp.float32)],
    )(p, t)

    stats = _sc_select()(d, kidx)

    out = pl.pallas_call(
        _body_c,
        out_specs=pl.BlockSpec((1, 1), lambda: (0, 0)),
        out_shape=jax.ShapeDtypeStruct((1, 1), jnp.float32),
    )(stats, nv)
    return out[0, 0]


# speedup vs baseline: 2.7469x; 1.0046x over previous
import functools

import jax
import jax.numpy as jnp
from jax import lax
from jax.experimental import pallas as pl
from jax.experimental.pallas import tpu as pltpu
from jax.experimental.pallas import tpu_sc as plsc

_LAMBDA_SSI = 0.5
_TOP_K_MASKING = 0.1
_EPS = 1e-06

_B = 16
_N = 384 * 384
_HALF = _N // 2
_NVEC = _HALF // 16
_C = 18432
_NCHUNK = _N // _C

_SHIFTS = (23, 15, 7, 0)
_NBINS = (256, 256, 256, 128)


def _body_a(p_ref, t_ref, d_ref, kidx_ref, nv_ref, nv_scr):
    i = pl.program_id(0)
    p = p_ref[...]
    t = t_ref[...]
    mask = (t > _EPS)
    d = jnp.log(jnp.maximum(p, _EPS)) - jnp.log(jnp.maximum(t, _EPS))
    d = jnp.where(mask, d, 0.0)
    d_ref[...] = d

    @pl.when(i == 0)
    def _init():
        nv_scr[...] = jnp.zeros_like(nv_scr)

    nv_scr[...] += jnp.sum(mask.astype(jnp.float32), axis=1, keepdims=True)

    @pl.when(i == _NCHUNK - 1)
    def _final():
        nv = nv_scr[...]
        k = (nv * _TOP_K_MASKING).astype(jnp.int32)
        k = jnp.minimum(k, _N - 1)
        kidx_ref[...] = jnp.broadcast_to(k, (_B, 128))
        nv_ref[...] = jnp.broadcast_to(nv, (_B, 8))


def _sc_body(d_hbm, kidx_hbm, out_hbm, d_buf, hist, pbuf, xveci, state,
             kbuf, stbuf, sh_hist, sh_x):
    c = lax.axis_index("c")
    s = lax.axis_index("s")
    row = c * 8 + s // 2
    half = s % 2
    is_scan = half == 0
    lanes = lax.iota(jnp.int32, 16)
    ones = jnp.ones((16,), jnp.float32)

    pltpu.sync_copy(d_hbm.at[row, pl.ds(half * _HALF, _HALF)], d_buf)

    @pl.when(is_scan)
    def _load_k():
        pltpu.sync_copy(kidx_hbm.at[row], kbuf)
        state[...] = jnp.full((16,), kbuf[pl.ds(0, 16)][0].astype(jnp.float32))

    def load_u(i):
        d = d_buf[pl.ds(i * 16, 16)]
        u = lax.bitcast_convert_type(d, jnp.int32) & 0x7FFFFFFF
        return d, u

    def unrolled(n, fn, unroll=16):
        plsc.parallel_loop(0, n, unroll=unroll)(fn)

    for p in range(4):
        shift = _SHIFTS[p]
        nb = _NBINS[p]

        def zfn(j):
            hist[pl.ds(j * 16, 16)] = jnp.zeros((16,), jnp.float32)
        unrolled(256, zfn)

        if p == 0:
            def sfn(i):
                _, u = load_u(i)
                b = lax.shift_right_logical(u, 23)
                plsc.addupdate_scatter(hist, [b * 16 + lanes], ones)
        else:
            pfx = xveci[...][0]
            pr_shift = _SHIFTS[p - 1]

            def sfn(i, pfx=pfx, shift=shift, pr_shift=pr_shift, mb=nb - 1):
                _, u = load_u(i)
                m = lax.shift_right_logical(u, pr_shift) == pfx
                b = lax.shift_right_logical(u, shift) & mb
                plsc.addupdate_scatter(hist, [b * 16 + lanes], ones, mask=m)
        unrolled(_NVEC, sfn)

        pltpu.sync_copy(hist, sh_hist.at[pl.ds(s * 4096, 4096)])
        plsc.subcore_barrier()

        @pl.when(is_scan)
        def _scan(p=p, nb=nb):
            pltpu.sync_copy(sh_hist.at[pl.ds((s + 1) * 4096, 4096)], pbuf)
            rank = state[...][0]

            init = (jnp.int32(0), jnp.int32(0), jnp.float32(0.0),
                    jnp.float32(0.0))

            @plsc.parallel_loop(0, nb, unroll=4, carry=init)
            def scbody(j, carry):
                found, bf, cgt, rn = carry
                b = (nb - 1) - j
                cb = (jnp.sum(hist[pl.ds(b * 16, 16)]) +
                      jnp.sum(pbuf[pl.ds(b * 16, 16)]))
                take = jnp.logical_and(found == 0, cgt + cb > rank)
                bf = jnp.where(take, b, bf)
                rn = jnp.where(take, rank - cgt, rn)
                found = jnp.where(take, 1, found)
                cgt = jnp.where(found > 0, cgt, cgt + cb)
                return found, bf, cgt, rn

            _, bf, _, rn = scbody
            state[...] = jnp.full((16,), rn)
            if p == 0:
                pfx_new = bf
            elif p < 3:
                pfx_new = xveci[...][0] * 256 + bf
            else:
                pfx_new = xveci[...][0] * 128 + bf
            xveci[...] = jnp.full((16,), pfx_new)
            pltpu.sync_copy(xveci, sh_x.at[pl.ds(s * 16, 16)])

        plsc.subcore_barrier()

        @pl.when(jnp.logical_not(is_scan))
        def _recv():
            pltpu.sync_copy(sh_x.at[pl.ds((s - 1) * 16, 16)], xveci)

    thr = xveci[...][0]

    z = jnp.zeros((16,), jnp.float32)

    @plsc.parallel_loop(0, _NVEC, unroll=16, carry=(z, z, z))
    def facc(i, carry):
        sv, ssv, cxv = carry
        d, u = load_u(i)
        m = u < thr
        dm = jnp.where(m, d, 0.0)
        return sv + dm, ssv + dm * dm, cxv + jnp.where(m, 0.0, 1.0)

    sv, ssv, cxv = facc
    stbuf[pl.ds(0, 16)] = sv
    stbuf[pl.ds(16, 16)] = ssv
    stbuf[pl.ds(32, 16)] = cxv
    pltpu.sync_copy(stbuf, out_hbm.at[half * 16 + row])


@functools.cache
def _sc_select():
    return pl.kernel(
        _sc_body,
        out_type=jax.ShapeDtypeStruct((2 * _B, 128), jnp.float32),
        mesh=plsc.VectorSubcoreMesh(core_axis_name="c", subcore_axis_name="s",
                                    num_cores=2, num_subcores=16),
        compiler_params=pltpu.CompilerParams(needs_layout_passes=False),
        scratch_types=[
        pltpu.VMEM((_HALF,), jnp.float32),
        pltpu.VMEM((4096,), jnp.float32),
        pltpu.VMEM((4096,), jnp.float32),
        pltpu.VMEM((16,), jnp.int32),
        pltpu.VMEM((16,), jnp.float32),
        pltpu.VMEM((128,), jnp.int32),
        pltpu.VMEM((128,), jnp.float32),
            pltpu.VMEM_SHARED((65536,), jnp.float32),
            pltpu.VMEM_SHARED((256,), jnp.int32),
        ],
    )


def _body_c(stats_ref, nv_ref, out_ref):
    st = stats_ref[0:_B, :] + stats_ref[_B:2 * _B, :]
    s = jnp.sum(st[:, 0:16], axis=1, keepdims=True)
    ss = jnp.sum(st[:, 16:32], axis=1, keepdims=True)
    cex = jnp.sum(st[:, 32:48], axis=1, keepdims=True)
    nv = nv_ref[:, 0:1]
    n = jnp.maximum(nv - cex, 1.0)
    v = ss / n - _LAMBDA_SSI * (s * s) / (n * n)
    row = jnp.sqrt(jnp.maximum(v, _EPS))
    out_ref[...] = jnp.mean(row).reshape(1, 1)


def kernel(prediction, target):
    p = prediction.reshape(_B, _N)
    t = target.reshape(_B, _N)
    d, kidx, nv = pl.pallas_call(
        _body_a,
        grid=(_NCHUNK,),
        in_specs=[
            pl.BlockSpec((_B, _C), lambda i: (0, i)),
            pl.BlockSpec((_B, _C), lambda i: (0, i)),
        ],
        out_specs=[
            pl.BlockSpec((_B, _C), lambda i: (0, i)),
            pl.BlockSpec((_B, 128), lambda i: (0, 0)),
            pl.BlockSpec((_B, 8), lambda i: (0, 0)),
        ],
        out_shape=[
            jax.ShapeDtypeStruct((_B, _N), jnp.float32),
            jax.ShapeDtypeStruct((_B, 128), jnp.int32),
            jax.ShapeDtypeStruct((_B, 8), jnp.float32),
        ],
        scratch_shapes=[pltpu.VMEM((_B, 1), jnp.float32)],
    )(p, t)

    stats = _sc_select()(d, kidx)

    out = pl.pallas_call(
        _body_c,
        out_specs=pl.BlockSpec((1, 1), lambda: (0, 0)),
        out_shape=jax.ShapeDtypeStruct((1, 1), jnp.float32),
    )(stats, nv)
    return out[0, 0]
